# Initial kernel scaffold; baseline (speedup 1.0000x reference)
import jax
import jax.numpy as jnp
from jax.experimental import pallas as pl


def kernel(x, edge_index, edge_weight):
    raise NotImplementedError("write your pallas kernel here")


# speedup vs baseline: 2.0985x; 2.0985x over previous
import functools

import jax
import jax.numpy as jnp
from jax import lax
from jax.experimental import pallas as pl
from jax.experimental.pallas import tpu as pltpu
from jax.experimental.pallas import tpu_sc as plsc

N = 10000
D = 128
E = 320000
MIN_NORM = 1e-15
EPS_F32 = 4e-3

NC = 2
NS = 16
L = 16
NW = NC * NS

CH = 128
CHUNKS = 80
EPW = CH * CHUNKS
E_PAD = EPW * NW
N_PAD = 10240
RPT = N_PAD // NS


def _logmap0_body(x_ref, o_ref):
    x = x_ref[...]
    n = jnp.maximum(jnp.sqrt(jnp.sum(x * x, axis=-1, keepdims=True)), MIN_NORM)
    t = jnp.clip(n, -1.0 + 1e-7, 1.0 - 1e-7)
    artanh = 0.5 * jnp.log((1.0 + t) / (1.0 - t))
    o_ref[...] = (artanh / n) * x


def _add2_body(a_ref, b_ref, o_ref):
    o_ref[...] = a_ref[...] + b_ref[...]


def _final_body(o1_ref, q0_ref, q1_ref, o_ref):
    u = o1_ref[...] + q0_ref[...] + q1_ref[...]
    n = jnp.maximum(jnp.sqrt(jnp.sum(u * u, axis=-1, keepdims=True)), MIN_NORM)
    g = jnp.tanh(n) * u / n
    gn = jnp.maximum(jnp.sqrt(jnp.sum(g * g, axis=-1, keepdims=True)), MIN_NORM)
    maxnorm = 1.0 - EPS_F32
    o_ref[...] = jnp.where(gn > maxnorm, g / gn * maxnorm, g)


_RB = 1000


def _row_spec():
    return pl.BlockSpec((_RB, D), lambda i: (i, 0))


def _tc_call(body, n_in):
    return pl.pallas_call(
        body,
        grid=(N // _RB,),
        in_specs=[_row_spec() for _ in range(n_in)],
        out_specs=_row_spec(),
        out_shape=jax.ShapeDtypeStruct((N, D), jnp.float32),
    )


def _spmm_sc(xt, rowp, colp, wp):
    mesh = plsc.VectorSubcoreMesh(core_axis_name="c", subcore_axis_name="s")

    @functools.partial(
        pl.kernel,
        out_type=(
            jax.ShapeDtypeStruct((N_PAD, D), jnp.float32),
            jax.ShapeDtypeStruct((N_PAD, D), jnp.float32),
        ),
        mesh=mesh,
        scratch_types=[
            pltpu.VMEM_SHARED((N_PAD, D), jnp.float32),
            pltpu.VMEM((CH, D), jnp.float32),
            pltpu.VMEM((CH,), jnp.int32),
            pltpu.VMEM((CH,), jnp.int32),
            pltpu.VMEM((CH, L), jnp.float32),
            pltpu.SemaphoreType.DMA,
        ],
    )
    def k(x_hbm, row_hbm, col_hbm, w_hbm, out0, out1, acc, rows_v, col_v,
          row_v, w_v, sem):
        cid = lax.axis_index("c")
        sid = lax.axis_index("s")
        wid = sid * NC + cid

        def zrow(j, _):
            for kk in range(D // L):
                rows_v[j, pl.ds(kk * L, L)] = jnp.zeros((L,), jnp.float32)
            return ()
        lax.fori_loop(0, CH, zrow, ())
        r0 = sid * RPT
        for zi in range(RPT // CH):
            pltpu.sync_copy(rows_v, acc.at[pl.ds(r0 + zi * CH, CH)])
        plsc.subcore_barrier()

        base = wid * EPW

        def chunk(i, _):
            off = pl.multiple_of(base + i * CH, CH)
            pltpu.sync_copy(col_hbm.at[pl.ds(off, CH)], col_v)
            pltpu.sync_copy(row_hbm.at[pl.ds(off, CH)], row_v)
            pltpu.sync_copy(w_hbm.at[pl.ds(off, CH)], w_v)
            pltpu.async_copy(x_hbm.at[col_v], rows_v, sem).wait()

            def scale(j, _):
                wj = w_v[j, :]
                for kk in range(D // L):
                    sl = pl.ds(kk * L, L)
                    rows_v[j, sl] = rows_v[j, sl] * wj
                return ()
            lax.fori_loop(0, CH, scale, ())

            pltpu.sync_copy(rows_v, acc.at[row_v], add=True)
            return ()

        lax.fori_loop(0, CHUNKS, chunk, ())
        plsc.subcore_barrier()

        @pl.when(cid == 0)
        def _():
            pltpu.sync_copy(acc.at[pl.ds(r0, RPT)], out0.at[pl.ds(r0, RPT)])

        @pl.when(cid == 1)
        def _():
            pltpu.sync_copy(acc.at[pl.ds(r0, RPT)], out1.at[pl.ds(r0, RPT)])

    return k(xt, rowp, colp, wp)


def kernel(x, edge_index, edge_weight):
    row = edge_index[0].astype(jnp.int32)
    col = edge_index[1].astype(jnp.int32)
    pad = E_PAD - E
    rowp = jnp.concatenate([row, jnp.zeros((pad,), jnp.int32)])
    colp = jnp.concatenate([col, jnp.zeros((pad,), jnp.int32)])
    wp = jnp.concatenate([edge_weight.astype(jnp.float32),
                          jnp.zeros((pad,), jnp.float32)])
    wp = jnp.broadcast_to(wp[:, None], (E_PAD, L))

    xt = _tc_call(_logmap0_body, 1)(x)
    p0, p1 = _spmm_sc(xt, rowp, colp, wp)
    out1 = _tc_call(_add2_body, 2)(p0, p1)
    q0, q1 = _spmm_sc(out1, rowp, colp, wp)
    return _tc_call(_final_body, 3)(out1, q0, q1)
